# Initial kernel scaffold; baseline (speedup 1.0000x reference)
import jax
import jax.numpy as jnp
from jax.experimental import pallas as pl


def kernel(x, W1, b1, g1, bt1, W2, b2, g2, bt2):
    raise NotImplementedError("write your pallas kernel here")


# speedup vs baseline: 6.2933x; 6.2933x over previous
import functools

import jax
import jax.numpy as jnp
from jax import lax
from jax.experimental import pallas as pl
from jax.experimental.pallas import tpu as pltpu
from jax.experimental.pallas import tpu_sc as plsc

KN = 16
NGROUPS = 32
EPS = 1e-5
NBLK_A = 256
NPB_D = 128


def _knn_uv_body(xf_ref, xb_ref, at_ref, bmt_ref, b1_ref,
                 idx_ref, u_ref, v_ref):
    b = pl.program_id(0)
    xf = xf_ref[0]
    xb = xb_ref[0]
    N = xf.shape[1]
    NB = xb.shape[1]
    cdims = (((0,), (0,)), ((), ()))
    inner = lax.dot_general(xb, xf, cdims,
                            preferred_element_type=jnp.float32)
    xx = jnp.sum(xf * xf, axis=0, keepdims=True)
    xxb = jnp.sum(xb * xb, axis=0, keepdims=True)
    dist = jnp.transpose(xxb) + xx - 2.0 * inner
    neg = -jnp.maximum(dist, 0.0)
    iota = lax.broadcasted_iota(jnp.int32, (NB, N), 1)
    cols = []
    for _ in range(KN):
        mx = jnp.max(neg, axis=1, keepdims=True)
        am = jnp.min(jnp.where(neg == mx, iota, N), axis=1, keepdims=True)
        cols.append(am)
        neg = jnp.where(iota == am, -jnp.inf, neg)
    idx_ref[0] = jnp.concatenate(cols, axis=1) + b * N
    u = lax.dot_general(xb, at_ref[...], cdims,
                        preferred_element_type=jnp.float32)
    u_ref[0] = u + b1_ref[...]
    v_ref[0] = lax.dot_general(xb, bmt_ref[...], cdims,
                               preferred_element_type=jnp.float32)


def _knn_uv(x, At, Bmt, b1r):
    B, C, N = x.shape
    grid = (B, N // NBLK_A)
    return pl.pallas_call(
        _knn_uv_body,
        grid=grid,
        in_specs=[
            pl.BlockSpec((1, C, N), lambda b, nb: (b, 0, 0)),
            pl.BlockSpec((1, C, NBLK_A), lambda b, nb: (b, 0, nb)),
            pl.BlockSpec((C, C), lambda b, nb: (0, 0)),
            pl.BlockSpec((C, C), lambda b, nb: (0, 0)),
            pl.BlockSpec((1, C), lambda b, nb: (0, 0)),
        ],
        out_specs=[
            pl.BlockSpec((1, NBLK_A, KN), lambda b, nb: (b, nb, 0)),
            pl.BlockSpec((1, NBLK_A, C), lambda b, nb: (b, nb, 0)),
            pl.BlockSpec((1, NBLK_A, C), lambda b, nb: (b, nb, 0)),
        ],
        out_shape=[
            jax.ShapeDtypeStruct((B, N, KN), jnp.int32),
            jax.ShapeDtypeStruct((B, N, C), jnp.float32),
            jax.ShapeDtypeStruct((B, N, C), jnp.float32),
        ],
    )(x, x, At, Bmt, b1r)


def _sc_gather(v_flat, idx_flat):
    R = idx_flat.shape[0]
    C = v_flat.shape[1]
    info = plsc.get_sparse_core_info()
    NW = info.num_cores * info.num_subcores
    per_w = R // NW
    CH = 128
    mesh = plsc.VectorSubcoreMesh(core_axis_name="c", subcore_axis_name="s")

    @functools.partial(
        pl.kernel,
        mesh=mesh,
        out_type=jax.ShapeDtypeStruct((R, C), jnp.float32),
        scratch_types=[
            pltpu.VMEM((CH,), jnp.int32),
            pltpu.VMEM((CH, C), jnp.float32),
            pltpu.SemaphoreType.DMA,
        ],
    )
    def gather_kernel(v_hbm, idx_hbm, out_hbm, idx_v, rows_v, sem):
        wid = lax.axis_index("s") * info.num_cores + lax.axis_index("c")
        base = wid * per_w

        def body(i, carry):
            off = base + i * CH
            pltpu.sync_copy(idx_hbm.at[pl.ds(off, CH)], idx_v)
            pltpu.async_copy(v_hbm.at[idx_v], rows_v, sem).wait()
            pltpu.sync_copy(rows_v, out_hbm.at[pl.ds(off, CH)])
            return carry

        lax.fori_loop(0, per_w // CH, body, 0)

    return gather_kernel(v_flat, idx_flat)


def _stats1_body(hg_ref, u_ref, out_ref):
    nb = pl.program_id(1)
    u = u_ref[0]
    NPB, C = u.shape
    h = hg_ref[0].reshape(NPB, KN, C) + u[:, None, :]
    s = jnp.sum(h, axis=(0, 1), keepdims=True).reshape(1, C)
    s2 = jnp.sum(h * h, axis=(0, 1), keepdims=True).reshape(1, C)
    st = jnp.concatenate([s, s2], axis=0)

    @pl.when(nb == 0)
    def _():
        out_ref[0] = st

    @pl.when(nb != 0)
    def _():
        out_ref[0] = out_ref[0] + st


def _stats1(hg4, u):
    B, NK, C = hg4.shape
    N = u.shape[1]
    grid = (B, N // NPB_D)
    return pl.pallas_call(
        _stats1_body,
        grid=grid,
        in_specs=[
            pl.BlockSpec((1, NPB_D * KN, C), lambda b, nb: (b, nb, 0)),
            pl.BlockSpec((1, NPB_D, C), lambda b, nb: (b, nb, 0)),
        ],
        out_specs=pl.BlockSpec((1, 2, C), lambda b, nb: (b, 0, 0)),
        out_shape=jax.ShapeDtypeStruct((B, 2, C), jnp.float32),
    )(hg4, u)


def _group_mats(C):
    ci = lax.broadcasted_iota(jnp.int32, (C, NGROUPS), 0) // (C // NGROUPS)
    gi = lax.broadcasted_iota(jnp.int32, (C, NGROUPS), 1)
    return (ci == gi).astype(jnp.float32)


def _mlp_body(hg_ref, u_ref, s1_ref, g1_ref, bt1_ref, w2t_ref, b2_ref,
              maxo_ref, mino_ref, s2o_ref, a1d1_ref):
    nb = pl.program_id(1)
    u = u_ref[0]
    NPB, C = u.shape
    N = pl.num_programs(1) * NPB
    cnt = float((C // NGROUPS)) * N * KN

    @pl.when(nb == 0)
    def _():
        G = _group_mats(C)
        s = s1_ref[0]
        sg = jnp.dot(s, G, preferred_element_type=jnp.float32)
        mean_g = sg[0:1] / cnt
        var_g = sg[1:2] / cnt - mean_g * mean_g
        rstd_g = lax.rsqrt(var_g + EPS)
        bc = jnp.dot(jnp.concatenate([mean_g, rstd_g], axis=0),
                     jnp.transpose(G), preferred_element_type=jnp.float32)
        a1 = g1_ref[...] * bc[1:2]
        d1 = bt1_ref[...] - bc[0:1] * a1
        a1d1_ref[...] = jnp.concatenate([a1, d1], axis=0)

    a1 = a1d1_ref[0:1]
    d1 = a1d1_ref[1:2]
    h3 = hg_ref[0].reshape(NPB, KN, C) + u[:, None, :]
    h2 = jnp.maximum(h3 * a1 + d1, 0.0)
    h2f = h2.reshape(NPB * KN, C)
    z = jnp.dot(h2f, w2t_ref[...],
                preferred_element_type=jnp.float32) + b2_ref[...]
    zs = jnp.sum(z, axis=0, keepdims=True)
    zs2 = jnp.sum(z * z, axis=0, keepdims=True)
    st = jnp.concatenate([zs, zs2], axis=0)

    @pl.when(nb == 0)
    def _():
        s2o_ref[0] = st

    @pl.when(nb != 0)
    def _():
        s2o_ref[0] = s2o_ref[0] + st

    z3 = z.reshape(NPB, KN, C)
    maxo_ref[0] = jnp.transpose(jnp.max(z3, axis=1))
    mino_ref[0] = jnp.transpose(jnp.min(z3, axis=1))


def _mlp(hg4, u, s1, g1r, bt1r, W2t, b2r):
    B, NK, C = hg4.shape
    N = u.shape[1]
    grid = (B, N // NPB_D)
    return pl.pallas_call(
        _mlp_body,
        grid=grid,
        in_specs=[
            pl.BlockSpec((1, NPB_D * KN, C), lambda b, nb: (b, nb, 0)),
            pl.BlockSpec((1, NPB_D, C), lambda b, nb: (b, nb, 0)),
            pl.BlockSpec((1, 2, C), lambda b, nb: (b, 0, 0)),
            pl.BlockSpec((1, C), lambda b, nb: (0, 0)),
            pl.BlockSpec((1, C), lambda b, nb: (0, 0)),
            pl.BlockSpec((C, C), lambda b, nb: (0, 0)),
            pl.BlockSpec((1, C), lambda b, nb: (0, 0)),
        ],
        out_specs=[
            pl.BlockSpec((1, C, NPB_D), lambda b, nb: (b, 0, nb)),
            pl.BlockSpec((1, C, NPB_D), lambda b, nb: (b, 0, nb)),
            pl.BlockSpec((1, 2, C), lambda b, nb: (b, 0, 0)),
        ],
        out_shape=[
            jax.ShapeDtypeStruct((B, C, N), jnp.float32),
            jax.ShapeDtypeStruct((B, C, N), jnp.float32),
            jax.ShapeDtypeStruct((B, 2, C), jnp.float32),
        ],
        scratch_shapes=[pltpu.VMEM((2, C), jnp.float32)],
    )(hg4, u, s1, g1r, bt1r, W2t, b2r)


def _final_body(mx_ref, mn_ref, s2_ref, g2_ref, bt2_ref, x_ref, y_ref):
    C, N = x_ref.shape[1], x_ref.shape[2]
    cnt = float((C // NGROUPS)) * N * KN
    G = _group_mats(C)
    s = s2_ref[0]
    sg = jnp.dot(s, G, preferred_element_type=jnp.float32)
    mean_g = sg[0:1] / cnt
    var_g = sg[1:2] / cnt - mean_g * mean_g
    rstd_g = lax.rsqrt(var_g + EPS)
    bc = jnp.dot(jnp.concatenate([mean_g, rstd_g], axis=0),
                 jnp.transpose(G), preferred_element_type=jnp.float32)
    a2 = g2_ref[...] * bc[1:2]
    d2 = bt2_ref[...] - bc[0:1] * a2
    ad = jnp.transpose(jnp.concatenate([a2, d2], axis=0))
    a2c = ad[:, 0:1]
    d2c = ad[:, 1:2]
    zm = mx_ref[0]
    zn = mn_ref[0]
    y = jnp.where(a2c > 0.0,
                  jnp.maximum(a2c * zm + d2c, 0.0),
                  jnp.maximum(a2c * zn + d2c, 0.0))
    y_ref[0] = y + x_ref[0]


def _final(mx, mn, s2, g2r, bt2r, x):
    B, C, N = x.shape
    return pl.pallas_call(
        _final_body,
        grid=(B,),
        in_specs=[
            pl.BlockSpec((1, C, N), lambda b: (b, 0, 0)),
            pl.BlockSpec((1, C, N), lambda b: (b, 0, 0)),
            pl.BlockSpec((1, 2, C), lambda b: (b, 0, 0)),
            pl.BlockSpec((1, C), lambda b: (0, 0)),
            pl.BlockSpec((1, C), lambda b: (0, 0)),
            pl.BlockSpec((1, C, N), lambda b: (b, 0, 0)),
        ],
        out_specs=pl.BlockSpec((1, C, N), lambda b: (b, 0, 0)),
        out_shape=jax.ShapeDtypeStruct((B, C, N), jnp.float32),
    )(mx, mn, s2, g2r, bt2r, x)


def kernel(x, W1, b1, g1, bt1, W2, b2, g2, bt2):
    B, C, N = x.shape
    At = jnp.transpose(W1[:, :C] - W1[:, C:])
    Bmt = jnp.transpose(W1[:, C:])
    idx, u, v = _knn_uv(x, At, Bmt, b1.reshape(1, C))
    hg = _sc_gather(v.reshape(B * N, C), idx.reshape(B * N * KN))
    hg4 = hg.reshape(B, N * KN, C)
    s1 = _stats1(hg4, u)
    mx, mn, s2 = _mlp(hg4, u, s1, g1.reshape(1, C), bt1.reshape(1, C),
                      jnp.transpose(W2), b2.reshape(1, C))
    return _final(mx, mn, s2, g2.reshape(1, C), bt2.reshape(1, C), x)


# speedup vs baseline: 7.0299x; 1.1170x over previous
import functools

import jax
import jax.numpy as jnp
from jax import lax
from jax.experimental import pallas as pl
from jax.experimental.pallas import tpu as pltpu
from jax.experimental.pallas import tpu_sc as plsc

KN = 16
NGROUPS = 32
EPS = 1e-5
NBLK_A = 256
NPB_D = 128


def _knn_uv_body(xf_ref, xb_ref, at_ref, bmt_ref, b1_ref,
                 idx_ref, u_ref, v_ref, us_ref):
    b = pl.program_id(0)
    nb = pl.program_id(1)
    xf = xf_ref[0]
    xb = xb_ref[0]
    N = xf.shape[1]
    cdims = (((0,), (0,)), ((), ()))
    inner = lax.dot_general(xb, xf, cdims,
                            preferred_element_type=jnp.float32)
    xx = jnp.sum(xf * xf, axis=0, keepdims=True)
    xxb = jnp.sum(xb * xb, axis=0, keepdims=True)
    dist = jnp.transpose(xxb) + xx - 2.0 * inner
    neg = -jnp.maximum(dist, 0.0)
    iota = lax.broadcasted_iota(jnp.int32, neg.shape, 1)
    cols = []
    for _ in range(KN):
        mx = jnp.max(neg, axis=1, keepdims=True)
        am = jnp.min(jnp.where(neg == mx, iota, N), axis=1, keepdims=True)
        cols.append(am)
        neg = jnp.where(iota == am, -jnp.inf, neg)
    idx_ref[0] = jnp.concatenate(cols, axis=1) + b * N
    u = lax.dot_general(xb, at_ref[...], cdims,
                        preferred_element_type=jnp.float32) + b1_ref[...]
    u_ref[0] = u
    v_ref[0] = lax.dot_general(xb, bmt_ref[...], cdims,
                               preferred_element_type=jnp.float32)
    us = jnp.sum(u, axis=0, keepdims=True)
    us2 = jnp.sum(u * u, axis=0, keepdims=True)
    st = jnp.concatenate([us, us2], axis=0)

    @pl.when(nb == 0)
    def _():
        us_ref[0] = st

    @pl.when(nb != 0)
    def _():
        us_ref[0] = us_ref[0] + st


def _knn_uv(x, At, Bmt, b1r):
    B, C, N = x.shape
    grid = (B, N // NBLK_A)
    return pl.pallas_call(
        _knn_uv_body,
        grid=grid,
        in_specs=[
            pl.BlockSpec((1, C, N), lambda b, nb: (b, 0, 0)),
            pl.BlockSpec((1, C, NBLK_A), lambda b, nb: (b, 0, nb)),
            pl.BlockSpec((C, C), lambda b, nb: (0, 0)),
            pl.BlockSpec((C, C), lambda b, nb: (0, 0)),
            pl.BlockSpec((1, C), lambda b, nb: (0, 0)),
        ],
        out_specs=[
            pl.BlockSpec((1, NBLK_A, KN), lambda b, nb: (b, nb, 0)),
            pl.BlockSpec((1, NBLK_A, C), lambda b, nb: (b, nb, 0)),
            pl.BlockSpec((1, NBLK_A, C), lambda b, nb: (b, nb, 0)),
            pl.BlockSpec((1, 2, C), lambda b, nb: (b, 0, 0)),
        ],
        out_shape=[
            jax.ShapeDtypeStruct((B, N, KN), jnp.int32),
            jax.ShapeDtypeStruct((B, N, C), jnp.float32),
            jax.ShapeDtypeStruct((B, N, C), jnp.float32),
            jax.ShapeDtypeStruct((B, 2, C), jnp.float32),
        ],
    )(x, x, At, Bmt, b1r)


def _sc_gather_stats(v_flat, idx_flat, u_flat):
    R = idx_flat.shape[0]
    C = v_flat.shape[1]
    info = plsc.get_sparse_core_info()
    NW = info.num_cores * info.num_subcores
    per_w = R // NW
    CH = 128
    NCHUNK = per_w // CH
    PPC = CH // KN
    CV = C // 16
    mesh = plsc.VectorSubcoreMesh(core_axis_name="c", subcore_axis_name="s")

    @functools.partial(
        pl.kernel,
        mesh=mesh,
        out_type=[
            jax.ShapeDtypeStruct((R, C), jnp.float32),
            jax.ShapeDtypeStruct((NW, 4, C), jnp.float32),
        ],
        scratch_types=[
            pltpu.VMEM((per_w,), jnp.int32),
            pltpu.VMEM((per_w // KN, C), jnp.float32),
            pltpu.VMEM((CH, C), jnp.float32),
            pltpu.VMEM((CH, C), jnp.float32),
            pltpu.VMEM((4, C), jnp.float32),
            pltpu.SemaphoreType.DMA,
            pltpu.SemaphoreType.DMA,
            pltpu.SemaphoreType.DMA,
            pltpu.SemaphoreType.DMA,
        ],
    )
    def gk(v_hbm, idx_hbm, u_hbm, out_hbm, part_hbm,
           idx_v, u_v, rows0, rows1, acc, g0, g1, w0, w1):
        wid = lax.axis_index("s") * info.num_cores + lax.axis_index("c")
        base = wid * per_w
        pltpu.sync_copy(idx_hbm.at[pl.ds(base, per_w)], idx_v)
        pltpu.sync_copy(u_hbm.at[pl.ds(wid * (per_w // KN), per_w // KN)],
                        u_v)
        zero = jnp.zeros((16,), jnp.float32)

        def zbody(c, carry):
            for r in range(4):
                acc[r, pl.ds(c * 16, 16)] = zero
            return carry

        lax.fori_loop(0, CV, zbody, 0)

        def g_src(chunk):
            return v_hbm.at[idx_v.at[pl.ds(chunk * CH, CH)]]

        def wb_dst(chunk):
            return out_hbm.at[pl.ds(base + chunk * CH, CH)]

        def compute(chunk, rows_ref):
            def pbody(p, carry):
                gp = chunk * PPC + p

                def cbody(c, carry2):
                    co = c * 16
                    r0 = rows_ref[p * KN, pl.ds(co, 16)]
                    s = r0
                    sq = r0 * r0
                    for j in range(1, KN):
                        rj = rows_ref[p * KN + j, pl.ds(co, 16)]
                        s = s + rj
                        sq = sq + rj * rj
                    uv = u_v[gp, pl.ds(co, 16)]
                    acc[0, pl.ds(co, 16)] = acc[0, pl.ds(co, 16)] + s
                    acc[1, pl.ds(co, 16)] = acc[1, pl.ds(co, 16)] + sq
                    acc[2, pl.ds(co, 16)] = acc[2, pl.ds(co, 16)] + uv * s
                    return carry2

                lax.fori_loop(0, CV, cbody, 0)
                return carry

            lax.fori_loop(0, PPC, pbody, 0)

        def step(chunk, rows_ref, gsem, wsem, prefetch):
            pltpu.make_async_copy(g_src(chunk), rows_ref, gsem).wait()
            wb = pltpu.async_copy(rows_ref, wb_dst(chunk), wsem)
            compute(chunk, rows_ref)
            wb.wait()
            if prefetch:
                pltpu.async_copy(g_src(chunk + 2), rows_ref, gsem)

        pltpu.async_copy(g_src(0), rows0, g0)
        pltpu.async_copy(g_src(1), rows1, g1)

        def lbody(i, carry):
            step(2 * i, rows0, g0, w0, True)
            step(2 * i + 1, rows1, g1, w1, True)
            return carry

        lax.fori_loop(0, NCHUNK // 2 - 1, lbody, 0)
        step(NCHUNK - 2, rows0, g0, w0, False)
        step(NCHUNK - 1, rows1, g1, w1, False)
        pltpu.sync_copy(acc, part_hbm.at[wid])

    return gk(v_flat, idx_flat, u_flat)


def _group_mats(C):
    ci = lax.broadcasted_iota(jnp.int32, (C, NGROUPS), 0) // (C // NGROUPS)
    gi = lax.broadcasted_iota(jnp.int32, (C, NGROUPS), 1)
    return (ci == gi).astype(jnp.float32)


def _mlp_body(hg_ref, u_ref, part_ref, us_ref, g1_ref, bt1_ref, w2t_ref,
              b2_ref, maxo_ref, mino_ref, s2o_ref, a1d1_ref):
    nb = pl.program_id(1)
    u = u_ref[0]
    NPB, C = u.shape
    N = pl.num_programs(1) * NPB
    cnt = float(C // NGROUPS) * N * KN

    @pl.when(nb == 0)
    def _():
        G = _group_mats(C)
        sp = jnp.sum(part_ref[0], axis=0)
        su = us_ref[0]
        s1 = 16.0 * su[0:1] + sp[0:1]
        s2 = 16.0 * su[1:2] + sp[1:2] + 2.0 * sp[2:3]
        s = jnp.concatenate([s1, s2], axis=0)
        sg = jnp.dot(s, G, preferred_element_type=jnp.float32)
        mean_g = sg[0:1] / cnt
        var_g = sg[1:2] / cnt - mean_g * mean_g
        rstd_g = lax.rsqrt(var_g + EPS)
        bc = jnp.dot(jnp.concatenate([mean_g, rstd_g], axis=0),
                     jnp.transpose(G), preferred_element_type=jnp.float32)
        a1 = g1_ref[...] * bc[1:2]
        d1 = bt1_ref[...] - bc[0:1] * a1
        a1d1_ref[...] = jnp.concatenate([a1, d1], axis=0)

    a1 = a1d1_ref[0:1]
    d1 = a1d1_ref[1:2]
    h3 = hg_ref[0].reshape(NPB, KN, C) + u[:, None, :]
    h2 = jnp.maximum(h3 * a1 + d1, 0.0)
    h2f = h2.reshape(NPB * KN, C)
    z = jnp.dot(h2f, w2t_ref[...],
                preferred_element_type=jnp.float32) + b2_ref[...]
    zs = jnp.sum(z, axis=0, keepdims=True)
    zs2 = jnp.sum(z * z, axis=0, keepdims=True)
    st = jnp.concatenate([zs, zs2], axis=0)

    @pl.when(nb == 0)
    def _():
        s2o_ref[0] = st

    @pl.when(nb != 0)
    def _():
        s2o_ref[0] = s2o_ref[0] + st

    z3 = z.reshape(NPB, KN, C)
    maxo_ref[0] = jnp.transpose(jnp.max(z3, axis=1))
    mino_ref[0] = jnp.transpose(jnp.min(z3, axis=1))


def _mlp(hg4, u, parts, usums, g1r, bt1r, W2t, b2r):
    B, NK, C = hg4.shape
    N = u.shape[1]
    WPB = parts.shape[1]
    grid = (B, N // NPB_D)
    return pl.pallas_call(
        _mlp_body,
        grid=grid,
        in_specs=[
            pl.BlockSpec((1, NPB_D * KN, C), lambda b, nb: (b, nb, 0)),
            pl.BlockSpec((1, NPB_D, C), lambda b, nb: (b, nb, 0)),
            pl.BlockSpec((1, WPB, 4, C), lambda b, nb: (b, 0, 0, 0)),
            pl.BlockSpec((1, 2, C), lambda b, nb: (b, 0, 0)),
            pl.BlockSpec((1, C), lambda b, nb: (0, 0)),
            pl.BlockSpec((1, C), lambda b, nb: (0, 0)),
            pl.BlockSpec((C, C), lambda b, nb: (0, 0)),
            pl.BlockSpec((1, C), lambda b, nb: (0, 0)),
        ],
        out_specs=[
            pl.BlockSpec((1, C, NPB_D), lambda b, nb: (b, 0, nb)),
            pl.BlockSpec((1, C, NPB_D), lambda b, nb: (b, 0, nb)),
            pl.BlockSpec((1, 2, C), lambda b, nb: (b, 0, 0)),
        ],
        out_shape=[
            jax.ShapeDtypeStruct((B, C, N), jnp.float32),
            jax.ShapeDtypeStruct((B, C, N), jnp.float32),
            jax.ShapeDtypeStruct((B, 2, C), jnp.float32),
        ],
        scratch_shapes=[pltpu.VMEM((2, C), jnp.float32)],
    )(hg4, u, parts, usums, g1r, bt1r, W2t, b2r)


def _final_body(mx_ref, mn_ref, s2_ref, g2_ref, bt2_ref, x_ref, y_ref):
    C, N = x_ref.shape[1], x_ref.shape[2]
    cnt = float(C // NGROUPS) * N * KN
    G = _group_mats(C)
    s = s2_ref[0]
    sg = jnp.dot(s, G, preferred_element_type=jnp.float32)
    mean_g = sg[0:1] / cnt
    var_g = sg[1:2] / cnt - mean_g * mean_g
    rstd_g = lax.rsqrt(var_g + EPS)
    bc = jnp.dot(jnp.concatenate([mean_g, rstd_g], axis=0),
                 jnp.transpose(G), preferred_element_type=jnp.float32)
    a2 = g2_ref[...] * bc[1:2]
    d2 = bt2_ref[...] - bc[0:1] * a2
    ad = jnp.transpose(jnp.concatenate([a2, d2], axis=0))
    a2c = ad[:, 0:1]
    d2c = ad[:, 1:2]
    zm = mx_ref[0]
    zn = mn_ref[0]
    y = jnp.where(a2c > 0.0,
                  jnp.maximum(a2c * zm + d2c, 0.0),
                  jnp.maximum(a2c * zn + d2c, 0.0))
    y_ref[0] = y + x_ref[0]


def _final(mx, mn, s2, g2r, bt2r, x):
    B, C, N = x.shape
    return pl.pallas_call(
        _final_body,
        grid=(B,),
        in_specs=[
            pl.BlockSpec((1, C, N), lambda b: (b, 0, 0)),
            pl.BlockSpec((1, C, N), lambda b: (b, 0, 0)),
            pl.BlockSpec((1, 2, C), lambda b: (b, 0, 0)),
            pl.BlockSpec((1, C), lambda b: (0, 0)),
            pl.BlockSpec((1, C), lambda b: (0, 0)),
            pl.BlockSpec((1, C, N), lambda b: (b, 0, 0)),
        ],
        out_specs=pl.BlockSpec((1, C, N), lambda b: (b, 0, 0)),
        out_shape=jax.ShapeDtypeStruct((B, C, N), jnp.float32),
    )(mx, mn, s2, g2r, bt2r, x)


def kernel(x, W1, b1, g1, bt1, W2, b2, g2, bt2):
    B, C, N = x.shape
    At = jnp.transpose(W1[:, :C] - W1[:, C:])
    Bmt = jnp.transpose(W1[:, C:])
    idx, u, v, usums = _knn_uv(x, At, Bmt, b1.reshape(1, C))
    hg, parts = _sc_gather_stats(v.reshape(B * N, C),
                                 idx.reshape(B * N * KN),
                                 u.reshape(B * N, C))
    NW = parts.shape[0]
    hg4 = hg.reshape(B, N * KN, C)
    parts4 = parts.reshape(B, NW // B, 4, C)
    mx, mn, s2 = _mlp(hg4, u, parts4, usums, g1.reshape(1, C),
                      bt1.reshape(1, C), jnp.transpose(W2), b2.reshape(1, C))
    return _final(mx, mn, s2, g2.reshape(1, C), bt2.reshape(1, C), x)
